# Initial kernel scaffold; baseline (speedup 1.0000x reference)
import jax
import jax.numpy as jnp
from jax.experimental import pallas as pl


def kernel(x, edge_index, edge_attr, W_embed, b_embed, W1, b1, W2, b2, Wp1, bp1, Wp2, bp2):
    raise NotImplementedError("write your pallas kernel here")


# speedup vs baseline: 6.2780x; 6.2780x over previous
import functools

import jax
import jax.numpy as jnp
from jax import lax
from jax.experimental import pallas as pl
from jax.experimental.pallas import tpu as pltpu
from jax.experimental.pallas import tpu_sc as plsc

NC = 2
NS = 16
NW = NC * NS
LANES = 16
NPAD = 10240
BM = 1024
K = 80

_MESH = dict(core_axis_name="c", subcore_axis_name="s", num_cores=NC,
             num_subcores=NS)


def _sc_degree(dst):
    (E,) = dst.shape
    ept = E // NW

    @functools.partial(
        pl.kernel,
        out_type=jax.ShapeDtypeStruct((NW, NPAD), jnp.float32),
        mesh=plsc.VectorSubcoreMesh(**_MESH),
        scratch_types=[
            pltpu.VMEM((ept + LANES,), jnp.int32),
            pltpu.VMEM((NPAD,), jnp.float32),
        ],
    )
    def k(dst_hbm, out_hbm, idx_v, hist_v):
        wid = lax.axis_index("s") * NC + lax.axis_index("c")
        pltpu.sync_copy(dst_hbm.at[pl.ds(wid * ept, ept)],
                        idx_v.at[pl.ds(0, ept)])

        def zero(i, carry):
            hist_v[pl.ds(i * LANES, LANES)] = jnp.zeros((LANES,), jnp.float32)
            return carry

        lax.fori_loop(0, NPAD // LANES, zero, 0)

        onehot = jnp.where(lax.iota(jnp.int32, LANES) == 0,
                           jnp.float32(1.0), jnp.float32(0.0))

        def body(e, carry):
            i = idx_v[pl.ds(e, LANES)][0]
            hist_v[pl.ds(i, LANES)] = hist_v[pl.ds(i, LANES)] + onehot
            return carry

        lax.fori_loop(0, ept, body, 0)
        pltpu.sync_copy(hist_v, out_hbm.at[wid])

    return k(dst)


def _sc_conv(g, src, dst, zeros_blk):
    (E,) = src.shape
    ept = E // NW
    H = g.shape[1]
    rows_per_tile = NPAD // NS

    @functools.partial(
        pl.kernel,
        out_type=jax.ShapeDtypeStruct((NC, NPAD, H), jnp.float32),
        mesh=plsc.VectorSubcoreMesh(**_MESH),
        scratch_types=[
            pltpu.VMEM((K,), jnp.int32),
            pltpu.VMEM((K,), jnp.int32),
            pltpu.VMEM((K, H), jnp.float32),
            pltpu.VMEM_SHARED((NPAD, H), jnp.float32),
            pltpu.SemaphoreType.DMA,
        ],
    )
    def k(g_hbm, src_hbm, dst_hbm, z_hbm, out_hbm, si_v, di_v, rows_v,
          acc_sh, sem):
        cid = lax.axis_index("c")
        sid = lax.axis_index("s")
        wid = sid * NC + cid
        pltpu.sync_copy(z_hbm, acc_sh.at[pl.ds(sid * rows_per_tile,
                                               rows_per_tile)])
        plsc.subcore_barrier()

        def chunk(j, carry):
            base = wid * ept + j * K
            pltpu.sync_copy(src_hbm.at[pl.ds(base, K)], si_v)
            pltpu.sync_copy(dst_hbm.at[pl.ds(base, K)], di_v)
            pltpu.async_copy(g_hbm.at[si_v], rows_v, sem).wait()
            pltpu.sync_copy(rows_v, acc_sh.at[di_v], add=True)
            return carry

        lax.fori_loop(0, ept // K, chunk, 0)
        plsc.subcore_barrier()
        pltpu.sync_copy(
            acc_sh.at[pl.ds(sid * rows_per_tile, rows_per_tile)],
            out_hbm.at[cid, pl.ds(sid * rows_per_tile, rows_per_tile)])

    return k(g, src, dst, zeros_blk)


def _sc_predict(A, B, row, col, eaf, de, wc, bp1, wp2, bp2pad):
    (E,) = row.shape
    ept = E // NW
    H = A.shape[1]
    DE = de
    HC = H // LANES

    @functools.partial(
        pl.kernel,
        out_type=jax.ShapeDtypeStruct((E,), jnp.float32),
        mesh=plsc.VectorSubcoreMesh(**_MESH),
        scratch_types=[
            pltpu.VMEM((K,), jnp.int32),
            pltpu.VMEM((K,), jnp.int32),
            pltpu.VMEM((K, H), jnp.float32),
            pltpu.VMEM((K, H), jnp.float32),
            pltpu.VMEM((K * DE + LANES,), jnp.float32),
            pltpu.VMEM((K,), jnp.float32),
            pltpu.VMEM((DE, H), jnp.float32),
            pltpu.VMEM((H,), jnp.float32),
            pltpu.VMEM((H,), jnp.float32),
            pltpu.VMEM((LANES,), jnp.float32),
            pltpu.SemaphoreType.DMA,
            pltpu.SemaphoreType.DMA,
        ],
    )
    def k(a_hbm, b_hbm, row_hbm, col_hbm, ea_hbm, wc_hbm, bp1_hbm,
          wp2_hbm, bp2_hbm, out_hbm, ri_v, ci_v, ar_v, br_v, ea_v, ob_v,
          wc_v, bp1_v, wp2_v, bp2_v, sem1, sem2):
        wid = lax.axis_index("s") * NC + lax.axis_index("c")
        pltpu.sync_copy(wc_hbm, wc_v)
        pltpu.sync_copy(bp1_hbm, bp1_v)
        pltpu.sync_copy(wp2_hbm, wp2_v)
        pltpu.sync_copy(bp2_hbm, bp2_v)
        b2v = bp2_v[pl.ds(0, LANES)]
        b2s = b2v[0]
        lane = lax.iota(jnp.int32, LANES)
        xor_idx = [jnp.bitwise_xor(lane, k) for k in (1, 2, 4, 8)]
        bp1c = [bp1_v[pl.ds(LANES * c, LANES)] for c in range(HC)]
        wp2c = [wp2_v[pl.ds(LANES * c, LANES)] for c in range(HC)]
        wcc = [[wc_v[j, pl.ds(LANES * c, LANES)] for c in range(HC)]
               for j in range(DE)]

        def chunk(j, carry):
            base = wid * ept + j * K
            pltpu.sync_copy(row_hbm.at[pl.ds(base, K)], ri_v)
            pltpu.sync_copy(col_hbm.at[pl.ds(base, K)], ci_v)
            pltpu.sync_copy(ea_hbm.at[pl.ds(base * DE, K * DE)],
                            ea_v.at[pl.ds(0, K * DE)])
            pltpu.async_copy(a_hbm.at[ri_v], ar_v, sem1).wait()
            pltpu.async_copy(b_hbm.at[ci_v], br_v, sem2).wait()

            def group(gidx, c2):
                res = jnp.zeros((LANES,), jnp.float32)
                for el in range(LANES):
                    e = gidx * LANES + el
                    ev = ea_v[pl.ds(e * DE, LANES)]
                    dacc = jnp.zeros((LANES,), jnp.float32)
                    for c in range(HC):
                        acc = bp1c[c]
                        for d in range(DE):
                            acc = acc + ev[d] * wcc[d][c]
                        acc = acc + ar_v[e, pl.ds(LANES * c, LANES)]
                        acc = acc + br_v[e, pl.ds(LANES * c, LANES)]
                        acc = jnp.maximum(acc, 0.0)
                        dacc = dacc + acc * wp2c[c]
                    for xi in xor_idx:
                        dacc = dacc + dacc.at[xi].get(
                            mode="promise_in_bounds", unique_indices=True)
                    res = jnp.where(lane == el, dacc + b2s, res)
                ob_v[pl.ds(gidx * LANES, LANES)] = res
                return c2

            lax.fori_loop(0, K // LANES, group, 0)
            pltpu.sync_copy(ob_v, out_hbm.at[pl.ds(base, K)])
            return carry

        lax.fori_loop(0, ept // K, chunk, 0)

    return k(A, B, row, col, eaf, wc, bp1, wp2, bp2pad)


def _tc_embed(x_pad, We, be, W1, hists):
    H = We.shape[1]

    def body(x_ref, we_ref, be_ref, w1_ref, h_ref, g_ref, dis_ref):
        h0 = jnp.maximum(
            jnp.dot(x_ref[...], we_ref[...],
                    preferred_element_type=jnp.float32) + be_ref[...], 0.0)
        deg = h_ref[0]
        for i in range(1, NW):
            deg = deg + h_ref[i]
        dis = lax.rsqrt(deg + 1.0)
        g_ref[...] = jnp.dot(h0, w1_ref[...],
                             preferred_element_type=jnp.float32) * dis
        dis_ref[...] = dis

    return pl.pallas_call(
        body,
        grid=(NPAD // BM,),
        in_specs=[
            pl.BlockSpec((BM, x_pad.shape[1]), lambda i: (i, 0)),
            pl.BlockSpec(We.shape, lambda i: (0, 0)),
            pl.BlockSpec((1, H), lambda i: (0, 0)),
            pl.BlockSpec(W1.shape, lambda i: (0, 0)),
            pl.BlockSpec((NW, BM, 1), lambda i: (0, i, 0)),
        ],
        out_specs=[
            pl.BlockSpec((BM, H), lambda i: (i, 0)),
            pl.BlockSpec((BM, 1), lambda i: (i, 0)),
        ],
        out_shape=[
            jax.ShapeDtypeStruct((NPAD, H), jnp.float32),
            jax.ShapeDtypeStruct((NPAD, 1), jnp.float32),
        ],
    )(x_pad, We, be, W1, hists)


def _tc_conv_combine(S, g, dis, b, W):
    H = g.shape[1]

    def body(s_ref, g_ref, dis_ref, b_ref, w_ref, out_ref):
        t = s_ref[0] + s_ref[1] + g_ref[...]
        h = jnp.maximum(dis_ref[...] * t + b_ref[...], 0.0)
        out_ref[...] = jnp.dot(
            h, w_ref[...], preferred_element_type=jnp.float32) * dis_ref[...]

    return pl.pallas_call(
        body,
        grid=(NPAD // BM,),
        in_specs=[
            pl.BlockSpec((NC, BM, H), lambda i: (0, i, 0)),
            pl.BlockSpec((BM, H), lambda i: (i, 0)),
            pl.BlockSpec((BM, 1), lambda i: (i, 0)),
            pl.BlockSpec((1, H), lambda i: (0, 0)),
            pl.BlockSpec((H, H), lambda i: (0, 0)),
        ],
        out_specs=pl.BlockSpec((BM, H), lambda i: (i, 0)),
        out_shape=jax.ShapeDtypeStruct((NPAD, H), jnp.float32),
    )(S, g, dis, b, W)


def _tc_final_tables(S, g, dis, b, Wa, Wb):
    H = g.shape[1]

    def body(s_ref, g_ref, dis_ref, b_ref, wa_ref, wb_ref, a_ref, bt_ref):
        t = s_ref[0] + s_ref[1] + g_ref[...]
        h = jnp.maximum(dis_ref[...] * t + b_ref[...], 0.0)
        a_ref[...] = jnp.dot(h, wa_ref[...],
                             preferred_element_type=jnp.float32)
        bt_ref[...] = jnp.dot(h, wb_ref[...],
                              preferred_element_type=jnp.float32)

    return pl.pallas_call(
        body,
        grid=(NPAD // BM,),
        in_specs=[
            pl.BlockSpec((NC, BM, H), lambda i: (0, i, 0)),
            pl.BlockSpec((BM, H), lambda i: (i, 0)),
            pl.BlockSpec((BM, 1), lambda i: (i, 0)),
            pl.BlockSpec((1, H), lambda i: (0, 0)),
            pl.BlockSpec((H, H), lambda i: (0, 0)),
            pl.BlockSpec((H, H), lambda i: (0, 0)),
        ],
        out_specs=[
            pl.BlockSpec((BM, H), lambda i: (i, 0)),
            pl.BlockSpec((BM, H), lambda i: (i, 0)),
        ],
        out_shape=[
            jax.ShapeDtypeStruct((NPAD, H), jnp.float32),
            jax.ShapeDtypeStruct((NPAD, H), jnp.float32),
        ],
    )(S, g, dis, b, Wa, Wb)


def kernel(x, edge_index, edge_attr, W_embed, b_embed, W1, b1, W2, b2,
           Wp1, bp1, Wp2, bp2):
    N, D = x.shape
    H = W1.shape[0]
    src = edge_index[0]
    dst = edge_index[1]

    x_pad = jnp.pad(x, ((0, NPAD - N), (0, 0)))
    zeros_blk = jnp.zeros((NPAD // NS, H), jnp.float32)

    hists = _sc_degree(dst)
    g1, dis = _tc_embed(x_pad, W_embed, b_embed[None], W1,
                        hists[..., None])
    S1 = _sc_conv(g1, src, dst, zeros_blk)
    g2 = _tc_conv_combine(S1, g1, dis, b1[None], W2)
    S2 = _sc_conv(g2, src, dst, zeros_blk)
    A, B = _tc_final_tables(S2, g2, dis, b2[None], Wp1[:H], Wp1[H:2 * H])
    out = _sc_predict(A, B, src, dst, edge_attr.reshape(-1),
                      edge_attr.shape[1], Wp1[2 * H:], bp1,
                      Wp2[:, 0], jnp.pad(bp2, (0, LANES - 1)))
    return out
